# baseline (speedup 1.0000x reference)
import functools

import jax
import jax.numpy as jnp
from jax import lax
from jax.experimental import pallas as pl
from jax.experimental.pallas import tpu as pltpu
from jax.experimental.pallas import tpu_sc as plsc

B = 16384
D = 64
NC = 2
NS = 16
NW = NC * NS
BPW = B // NW
L = 16
WAVES = BPW // 2


DEPTH = 7
MAIN = (BPW // DEPTH) * DEPTH


def _mf_body(user_hbm, item_hbm, ut_hbm, it_hbm, out_hbm,
             uidx, iidx,
             ub0, ub1, ub2, ub3, ub4, ub5, ub6,
             vb0, vb1, vb2, vb3, vb4, vb5, vb6,
             outv, sem0, sem1, sem2, sem3, sem4, sem5, sem6):
    wid = lax.axis_index("s") * NC + lax.axis_index("c")
    base = wid * BPW

    ub = (ub0, ub1, ub2, ub3, ub4, ub5, ub6)
    vb = (vb0, vb1, vb2, vb3, vb4, vb5, vb6)
    sems = (sem0, sem1, sem2, sem3, sem4, sem5, sem6)

    pltpu.sync_copy(user_hbm.at[pl.ds(base, BPW)], uidx.at[pl.ds(0, BPW)])
    pltpu.sync_copy(item_hbm.at[pl.ds(base, BPW)], iidx.at[pl.ds(0, BPW)])

    lanes = lax.iota(jnp.int32, L)
    zeros = jnp.zeros((L,), jnp.float32)

    def fire(w, parity):
        i0 = jnp.minimum(w, BPW - 1)
        uv = uidx[pl.ds(i0, L)]
        iv = iidx[pl.ds(i0, L)]
        cu = pl.multiple_of((uv[0] >> 7) * 128, 128)
        cv = pl.multiple_of((iv[0] >> 7) * 128, 128)
        pltpu.async_copy(ut_hbm.at[:, pl.ds(cu, 128)], ub[parity],
                         sems[parity])
        pltpu.async_copy(it_hbm.at[:, pl.ds(cv, 128)], vb[parity],
                         sems[parity])

    for s in range(DEPTH):
        fire(s, s)

    def do_wave(w, q, s, refire):
        for _ in range(2):
            pltpu.make_async_copy(ut_hbm.at[:, pl.ds(0, 128)],
                                  ub[0], sems[s]).wait()
        uv = uidx[pl.ds(w, L)]
        iv = iidx[pl.ds(w, L)]
        lu = jnp.full((L,), uv[0] & 127, jnp.int32)
        lv = jnp.full((L,), iv[0] & 127, jnp.int32)
        acc = None
        for j in range(D // L):
            rows = lanes + (j * L)
            uc = plsc.load_gather(ub[s], [rows, lu])
            vc = plsc.load_gather(vb[s], [rows, lv])
            prod = uc * vc
            acc = prod if acc is None else acc + prod
        q = jnp.where(lanes == (w & 15), jnp.sum(acc), q)
        if refire:
            fire(w + DEPTH, s)
        flush = (w & 15) == 15
        @pl.when(flush)
        def _():
            outv[pl.ds((w >> 4) * L, L)] = 1.0 / (1.0 + jnp.exp(-q))
        return jnp.where(flush, zeros, q)

    def group_body(t, q):
        for s in range(DEPTH):
            q = do_wave(DEPTH * t + s, q, s, True)
        return q

    q = lax.fori_loop(0, MAIN // DEPTH, group_body, zeros)
    for w in range(MAIN, BPW):
        q = do_wave(jnp.int32(w), q, w % DEPTH, False)

    for w in range(BPW, MAIN + DEPTH):
        for _ in range(2):
            pltpu.make_async_copy(ut_hbm.at[:, pl.ds(0, 128)],
                                  ub[0], sems[w % DEPTH]).wait()

    pltpu.sync_copy(outv, out_hbm.at[pl.ds(base, BPW)])


def kernel(user, item, user_table, item_table):
    mesh = plsc.VectorSubcoreMesh(core_axis_name="c", subcore_axis_name="s")
    blk = lambda: pltpu.VMEM((D, 128), jnp.float32)
    run = functools.partial(
        pl.kernel,
        out_type=jax.ShapeDtypeStruct((B,), jnp.float32),
        mesh=mesh,
        compiler_params=pltpu.CompilerParams(needs_layout_passes=False),
        scratch_types=[
            pltpu.VMEM((BPW + L,), jnp.int32),
            pltpu.VMEM((BPW + L,), jnp.int32),
        ] + [blk() for _ in range(2 * DEPTH)] + [
            pltpu.VMEM((BPW,), jnp.float32),
        ] + [pltpu.SemaphoreType.DMA] * DEPTH,
    )(_mf_body)
    return run(user, item, user_table.T, item_table.T)


# speedup vs baseline: 1.0049x; 1.0049x over previous
import functools

import jax
import jax.numpy as jnp
from jax import lax
from jax.experimental import pallas as pl
from jax.experimental.pallas import tpu as pltpu
from jax.experimental.pallas import tpu_sc as plsc

B = 16384
D = 64
NC = 2
NS = 16
NW = NC * NS
BPW = B // NW
L = 16
WAVES = BPW // 2


DEPTH = 6
MAIN = (BPW // DEPTH) * DEPTH


def _mf_body(user_hbm, item_hbm, ut_hbm, it_hbm, out_hbm,
             uidx, iidx,
             ub0, ub1, ub2, ub3, ub4, ub5,
             vb0, vb1, vb2, vb3, vb4, vb5,
             outv, sem0, sem1, sem2, sem3, sem4, sem5):
    wid = lax.axis_index("s") * NC + lax.axis_index("c")
    base = wid * BPW

    ub = (ub0, ub1, ub2, ub3, ub4, ub5)
    vb = (vb0, vb1, vb2, vb3, vb4, vb5)
    sems = (sem0, sem1, sem2, sem3, sem4, sem5)

    pltpu.sync_copy(user_hbm.at[pl.ds(base, BPW)], uidx.at[pl.ds(0, BPW)])
    pltpu.sync_copy(item_hbm.at[pl.ds(base, BPW)], iidx.at[pl.ds(0, BPW)])

    lanes = lax.iota(jnp.int32, L)
    zeros = jnp.zeros((L,), jnp.float32)

    def fire(w, parity):
        i0 = jnp.minimum(w, BPW - 1)
        uv = uidx[pl.ds(i0, L)]
        iv = iidx[pl.ds(i0, L)]
        cu = pl.multiple_of((uv[0] >> 7) * 128, 128)
        cv = pl.multiple_of((iv[0] >> 7) * 128, 128)
        pltpu.async_copy(ut_hbm.at[:, pl.ds(cu, 128)], ub[parity],
                         sems[parity])
        pltpu.async_copy(it_hbm.at[:, pl.ds(cv, 128)], vb[parity],
                         sems[parity])

    for s in range(DEPTH):
        fire(s, s)

    def do_wave(w, q, s, refire):
        for _ in range(2):
            pltpu.make_async_copy(ut_hbm.at[:, pl.ds(0, 128)],
                                  ub[0], sems[s]).wait()
        uv = uidx[pl.ds(w, L)]
        iv = iidx[pl.ds(w, L)]
        lu = jnp.full((L,), uv[0] & 127, jnp.int32)
        lv = jnp.full((L,), iv[0] & 127, jnp.int32)
        acc = None
        for j in range(D // L):
            rows = lanes + (j * L)
            uc = plsc.load_gather(ub[s], [rows, lu])
            vc = plsc.load_gather(vb[s], [rows, lv])
            prod = uc * vc
            acc = prod if acc is None else acc + prod
        q = jnp.where(lanes == (w & 15), jnp.sum(acc), q)
        if refire:
            fire(w + DEPTH, s)
        flush = (w & 15) == 15
        @pl.when(flush)
        def _():
            outv[pl.ds((w >> 4) * L, L)] = 1.0 / (1.0 + jnp.exp(-q))
        return jnp.where(flush, zeros, q)

    def group_body(t, q):
        for s in range(DEPTH):
            q = do_wave(DEPTH * t + s, q, s, True)
        return q

    q = lax.fori_loop(0, MAIN // DEPTH, group_body, zeros)
    for w in range(MAIN, BPW):
        q = do_wave(jnp.int32(w), q, w % DEPTH, False)

    for w in range(BPW, MAIN + DEPTH):
        for _ in range(2):
            pltpu.make_async_copy(ut_hbm.at[:, pl.ds(0, 128)],
                                  ub[0], sems[w % DEPTH]).wait()

    pltpu.sync_copy(outv, out_hbm.at[pl.ds(base, BPW)])


def kernel(user, item, user_table, item_table):
    mesh = plsc.VectorSubcoreMesh(core_axis_name="c", subcore_axis_name="s")
    blk = lambda: pltpu.VMEM((D, 128), jnp.float32)
    run = functools.partial(
        pl.kernel,
        out_type=jax.ShapeDtypeStruct((B,), jnp.float32),
        mesh=mesh,
        compiler_params=pltpu.CompilerParams(needs_layout_passes=False),
        scratch_types=[
            pltpu.VMEM((BPW + L,), jnp.int32),
            pltpu.VMEM((BPW + L,), jnp.int32),
        ] + [blk() for _ in range(2 * DEPTH)] + [
            pltpu.VMEM((BPW,), jnp.float32),
        ] + [pltpu.SemaphoreType.DMA] * DEPTH,
    )(_mf_body)
    return run(user, item, user_table.T, item_table.T)
